# Initial kernel scaffold; baseline (speedup 1.0000x reference)
import jax
import jax.numpy as jnp
from jax.experimental import pallas as pl


def kernel(hidden_states, router_weight, e_score_correction_bias, expert_up, expert_down, shared_up, shared_down):
    raise NotImplementedError("write your pallas kernel here")


# speedup vs baseline: 1.1430x; 1.1430x over previous
import functools

import jax
import jax.numpy as jnp
from jax import lax
from jax.experimental import pallas as pl
from jax.experimental.pallas import tpu as pltpu
from jax.experimental.pallas import tpu_sc as plsc

H = 2048
FF_MOE = 1024
FF_SHARED = 2048
E = 8
T = 4096
TBLK = 1024
NBLK = T // TBLK
TB = 128
P_PAD = 2 * T + E * TB
NB = P_PAD // TB
SCALE = 2.5
F32 = jnp.float32
I32 = jnp.int32

NC = 2
NS = 16
NW = NC * NS
RPW = P_PAD // NW
CH = 32
NCH = RPW // CH
TPW = T // NW
CC = 16
NCC = TPW // CC


def _router_kernel(x_ref, rw_ref, b_ref, pos0_ref, pos1_ref, w0_ref, w1_ref, meta_ref,
                   idx0_s, idx1_s, r0_s, r1_s, c0_s, c1_s):
    i = pl.program_id(0)

    @pl.when(i < NBLK)
    def _block():
        x = x_ref[...]
        rw = rw_ref[...]
        logits = lax.dot_general(x, rw, (((1,), (1,)), ((), ())),
                                 preferred_element_type=F32)
        scores = jax.nn.sigmoid(logits)
        sfc = scores + b_ref[...]
        gcols = [sfc[:, 2 * g:2 * g + 1] + sfc[:, 2 * g + 1:2 * g + 2] for g in range(4)]
        gmask = []
        for g in range(4):
            beat = jnp.zeros((TBLK, 1), F32)
            for g2 in range(4):
                if g2 == g:
                    continue
                if g2 < g:
                    beat += (gcols[g2] >= gcols[g]).astype(F32)
                else:
                    beat += (gcols[g2] > gcols[g]).astype(F32)
            gmask.append(beat < 2.0)
        mcols = [jnp.where(gmask[e // 2], sfc[:, e:e + 1], 0.0) for e in range(E)]
        eranks = []
        for e in range(E):
            beat = jnp.zeros((TBLK, 1), F32)
            for e2 in range(E):
                if e2 == e:
                    continue
                if e2 < e:
                    beat += (mcols[e2] >= mcols[e]).astype(F32)
                else:
                    beat += (mcols[e2] > mcols[e]).astype(F32)
            eranks.append(beat)
        oh0 = [(eranks[e] == 0.0).astype(F32) for e in range(E)]
        oh1 = [(eranks[e] == 1.0).astype(F32) for e in range(E)]
        w0 = jnp.zeros((TBLK, 1), F32)
        w1 = jnp.zeros((TBLK, 1), F32)
        idx0 = jnp.zeros((TBLK, 1), F32)
        idx1 = jnp.zeros((TBLK, 1), F32)
        for e in range(E):
            w0 += oh0[e] * scores[:, e:e + 1]
            w1 += oh1[e] * scores[:, e:e + 1]
            idx0 += oh0[e] * float(e)
            idx1 += oh1[e] * float(e)
        denom = w0 + w1 + 1e-20
        sl = pl.ds(i * TBLK, TBLK)
        w0_ref[sl, :] = w0 / denom * SCALE
        w1_ref[sl, :] = w1 / denom * SCALE
        idx0_s[sl, :] = idx0
        idx1_s[sl, :] = idx1
        rows = lax.broadcasted_iota(I32, (TBLK, TBLK), 0)
        cols = lax.broadcasted_iota(I32, (TBLK, TBLK), 1)
        lmat = (cols < rows).astype(F32)
        lane = lax.broadcasted_iota(I32, (1, 2 * E), 1)
        oh = jnp.zeros((TBLK, 2 * E), F32)
        for e in range(E):
            oh += oh0[e] * (lane == e).astype(F32)
            oh += oh1[e] * (lane == E + e).astype(F32)
        s = lax.dot_general(lmat, oh, (((1,), (0,)), ((), ())),
                            preferred_element_type=F32)
        r0 = jnp.zeros((TBLK, 1), F32)
        r1 = jnp.zeros((TBLK, 1), F32)
        for e in range(E):
            r0 += oh0[e] * s[:, e:e + 1]
            r1 += oh1[e] * s[:, E + e:E + e + 1]
        r0_s[sl, :] = r0
        r1_s[sl, :] = r1
        for e in range(E):
            c0_s[pl.ds(i, 1), pl.ds(e, 1)] = jnp.sum(oh0[e], axis=0, keepdims=True)
            c1_s[pl.ds(i, 1), pl.ds(e, 1)] = jnp.sum(oh1[e], axis=0, keepdims=True)

    @pl.when(i == NBLK)
    def _final():
        c0 = c0_s[...]
        c1 = c1_s[...]
        ct0 = jnp.sum(c0, axis=0, keepdims=True)
        ct = ct0 + jnp.sum(c1, axis=0, keepdims=True)
        cti = ct.astype(I32)
        pc = ((cti + (TB - 1)) // TB) * TB
        base = []
        acc = jnp.zeros((1, 1), I32)
        for e in range(E):
            base.append(acc)
            acc = acc + pc[:, e:e + 1]
        rs = lax.broadcasted_iota(I32, (1, 128), 1) * TB
        meta = jnp.zeros((1, 128), I32)
        for e in range(E):
            active = (rs >= base[e]) & (rs < base[e] + cti[:, e:e + 1])
            meta += (e + 1) * active.astype(I32)
        meta_ref[...] = meta - 1
        for bl in range(NBLK):
            sl = pl.ds(bl * TBLK, TBLK)
            idx0 = idx0_s[sl, :]
            idx1 = idx1_s[sl, :]
            pos0 = r0_s[sl, :].astype(I32)
            pos1 = r1_s[sl, :].astype(I32)
            for e in range(E):
                prior0 = base[e]
                prior1 = base[e] + ct0[:, e:e + 1].astype(I32)
                for bl2 in range(bl):
                    prior0 = prior0 + c0[bl2:bl2 + 1, e:e + 1].astype(I32)
                    prior1 = prior1 + c1[bl2:bl2 + 1, e:e + 1].astype(I32)
                fe = float(e)
                pos0 += jnp.where(idx0 == fe, prior0, 0)
                pos1 += jnp.where(idx1 == fe, prior1, 0)
            pos0_ref[sl, :] = pos0
            pos1_ref[sl, :] = pos1


def _router_tc(flat, rw, bias2):
    return pl.pallas_call(
        _router_kernel,
        grid=(NBLK + 1,),
        in_specs=[
            pl.BlockSpec((TBLK, H), lambda i: (jnp.minimum(i, NBLK - 1), 0)),
            pl.BlockSpec((E, H), lambda i: (0, 0)),
            pl.BlockSpec((1, E), lambda i: (0, 0)),
        ],
        out_specs=[
            pl.BlockSpec((T, 1), lambda i: (0, 0)),
            pl.BlockSpec((T, 1), lambda i: (0, 0)),
            pl.BlockSpec((T, 1), lambda i: (0, 0)),
            pl.BlockSpec((T, 1), lambda i: (0, 0)),
            pl.BlockSpec((1, 128), lambda i: (0, 0)),
        ],
        out_shape=[
            jax.ShapeDtypeStruct((T, 1), I32),
            jax.ShapeDtypeStruct((T, 1), I32),
            jax.ShapeDtypeStruct((T, 1), F32),
            jax.ShapeDtypeStruct((T, 1), F32),
            jax.ShapeDtypeStruct((1, 128), I32),
        ],
        scratch_shapes=[
            pltpu.VMEM((T, 1), F32), pltpu.VMEM((T, 1), F32),
            pltpu.VMEM((T, 1), F32), pltpu.VMEM((T, 1), F32),
            pltpu.VMEM((NBLK, E), F32), pltpu.VMEM((NBLK, E), F32),
        ],
    )(flat, rw, bias2)


_SC_MESH = plsc.VectorSubcoreMesh(core_axis_name="c", subcore_axis_name="s")


@functools.partial(
    pl.kernel,
    mesh=_SC_MESH,
    compiler_params=pltpu.CompilerParams(needs_layout_passes=False),
    out_type=(
        jax.ShapeDtypeStruct((P_PAD, H), F32),
        jax.ShapeDtypeStruct((P_PAD,), F32),
    ),
    scratch_types=[
        pltpu.VMEM((2 * T,), I32),
        pltpu.VMEM((2 * T,), F32),
        pltpu.VMEM((P_PAD,), I32),
        pltpu.VMEM((P_PAD,), F32),
        pltpu.VMEM((CH,), I32),
        pltpu.VMEM((CH, H), F32),
        pltpu.SemaphoreType.DMA,
    ],
)
def _dispatch_sc(pos0_hbm, pos1_hbm, w0_hbm, w1_hbm, flat_hbm, xs_hbm, ws_hbm,
                 pos_v, w_v, tok_v, wsv, idx_v, rows_v, sem):
    cid = lax.axis_index("c")
    sid = lax.axis_index("s")
    wid = sid * NC + cid
    pltpu.sync_copy(pos0_hbm, pos_v.at[pl.ds(0, T)])
    pltpu.sync_copy(pos1_hbm, pos_v.at[pl.ds(T, T)])
    pltpu.sync_copy(w0_hbm, w_v.at[pl.ds(0, T)])
    pltpu.sync_copy(w1_hbm, w_v.at[pl.ds(T, T)])

    zi = jnp.zeros((16,), I32)
    zf = jnp.zeros((16,), F32)

    def initbody(j, c):
        tok_v[pl.ds(j * 16, 16)] = zi
        wsv[pl.ds(j * 16, 16)] = zf
        return c

    lax.fori_loop(0, P_PAD // 16, initbody, 0)

    lanes = lax.broadcasted_iota(I32, (16,), 0)

    def scbody(j, c):
        pvec = pos_v[pl.ds(j * 16, 16)]
        wvec = w_v[pl.ds(j * 16, 16)]
        tvec = lax.rem(j * 16, T) + lanes
        plsc.store_scatter(tok_v, [pvec], tvec)
        plsc.store_scatter(wsv, [pvec], wvec)
        return c

    lax.fori_loop(0, 2 * T // 16, scbody, 0)

    pltpu.sync_copy(wsv.at[pl.ds(wid * RPW, RPW)], ws_hbm.at[pl.ds(wid * RPW, RPW)])

    for c in range(NCH):
        start = wid * RPW + c * CH
        for hh in range(CH // 16):
            idx_v[pl.ds(hh * 16, 16)] = tok_v[pl.ds(start + hh * 16, 16)]
        pltpu.async_copy(flat_hbm.at[idx_v], rows_v, sem).wait()
        pltpu.sync_copy(rows_v, xs_hbm.at[pl.ds(start, CH)])


def _moe_kernel(meta_ref, x_ref, up_ref, dn_ref, w_ref, y_ref):
    b = pl.program_id(0)
    e = meta_ref[b]

    @pl.when(e >= 0)
    def _():
        x = x_ref[...]
        u = jnp.squeeze(up_ref[...], 0)
        h = lax.dot_general(x, u, (((1,), (1,)), ((), ())),
                            preferred_element_type=F32)
        h = jnp.maximum(h, 0.0)
        h = h * h
        d = jnp.squeeze(dn_ref[...], 0)
        y = lax.dot_general(h, d, (((1,), (1,)), ((), ())),
                            preferred_element_type=F32)
        y_ref[...] = y * w_ref[...]


def _moe_tc(meta_nb, xs, expert_up, expert_down, ws_col):
    grid_spec = pltpu.PrefetchScalarGridSpec(
        num_scalar_prefetch=1,
        grid=(NB,),
        in_specs=[
            pl.BlockSpec((TB, H), lambda b, m: (b, 0)),
            pl.BlockSpec((1, FF_MOE, H), lambda b, m: (jnp.maximum(m[b], 0), 0, 0)),
            pl.BlockSpec((1, H, FF_MOE), lambda b, m: (jnp.maximum(m[b], 0), 0, 0)),
            pl.BlockSpec((TB, 1), lambda b, m: (b, 0)),
        ],
        out_specs=pl.BlockSpec((TB, H), lambda b, m: (b, 0)),
    )
    return pl.pallas_call(
        _moe_kernel,
        grid_spec=grid_spec,
        out_shape=jax.ShapeDtypeStruct((P_PAD, H), F32),
        compiler_params=pltpu.CompilerParams(vmem_limit_bytes=100 * 1024 * 1024),
    )(meta_nb, xs, expert_up, expert_down, ws_col)


def _shared_kernel(x_ref, su_ref, sd_ref, o_ref):
    x = x_ref[...]
    h = lax.dot_general(x, su_ref[...], (((1,), (1,)), ((), ())),
                        preferred_element_type=F32)
    h = jnp.maximum(h, 0.0)
    h = h * h
    o_ref[...] = lax.dot_general(h, sd_ref[...], (((1,), (1,)), ((), ())),
                                 preferred_element_type=F32)


def _shared_tc(flat, su, sd):
    sb = 256
    return pl.pallas_call(
        _shared_kernel,
        grid=(T // sb,),
        in_specs=[
            pl.BlockSpec((sb, H), lambda i: (i, 0)),
            pl.BlockSpec((FF_SHARED, H), lambda i: (0, 0)),
            pl.BlockSpec((H, FF_SHARED), lambda i: (0, 0)),
        ],
        out_specs=pl.BlockSpec((sb, H), lambda i: (i, 0)),
        out_shape=jax.ShapeDtypeStruct((T, H), F32),
        compiler_params=pltpu.CompilerParams(vmem_limit_bytes=100 * 1024 * 1024),
    )(flat, su, sd)


@functools.partial(
    pl.kernel,
    mesh=_SC_MESH,
    compiler_params=pltpu.CompilerParams(needs_layout_passes=False),
    out_type=jax.ShapeDtypeStruct((T, H), F32),
    scratch_types=[
        pltpu.VMEM((CC,), I32),
        pltpu.VMEM((CC,), I32),
        pltpu.VMEM((CC, H), F32),
        pltpu.VMEM((CC, H), F32),
        pltpu.VMEM((CC, H), F32),
        pltpu.SemaphoreType.DMA,
        pltpu.SemaphoreType.DMA,
    ],
)
def _combine_sc(ys_hbm, sh_hbm, pos0_hbm, pos1_hbm, out_hbm,
                i0, i1, r0, r1, accv, s0, s1):
    cid = lax.axis_index("c")
    sid = lax.axis_index("s")
    wid = sid * NC + cid
    for c in range(NCC):
        base = wid * TPW + c * CC
        pltpu.sync_copy(pos0_hbm.at[pl.ds(base, CC)], i0)
        pltpu.sync_copy(pos1_hbm.at[pl.ds(base, CC)], i1)
        cp0 = pltpu.async_copy(ys_hbm.at[i0], r0, s0)
        cp1 = pltpu.async_copy(ys_hbm.at[i1], r1, s1)
        pltpu.sync_copy(sh_hbm.at[pl.ds(base, CC)], accv)
        cp0.wait()
        cp1.wait()
        for t in range(CC):
            def body(v, carry):
                for u in range(8):
                    sl = pl.ds(v * 128 + u * 16, 16)
                    accv[t, sl] = accv[t, sl] + r0[t, sl] + r1[t, sl]
                return carry

            lax.fori_loop(0, H // 128, body, 0)
        pltpu.sync_copy(accv, out_hbm.at[pl.ds(base, CC)])


def kernel(hidden_states, router_weight, e_score_correction_bias,
           expert_up, expert_down, shared_up, shared_down):
    orig_shape = hidden_states.shape
    flat = hidden_states.reshape(-1, H)
    bias2 = e_score_correction_bias.reshape(1, E)
    pos0, pos1, w0, w1, meta = _router_tc(flat, router_weight, bias2)
    pos0 = pos0.reshape(T)
    pos1 = pos1.reshape(T)
    meta_nb = meta.reshape(128)[:NB]
    xs, ws = _dispatch_sc(pos0, pos1, w0.reshape(T), w1.reshape(T), flat)
    ys = _moe_tc(meta_nb, xs, expert_up, expert_down, ws.reshape(P_PAD, 1))
    sh = _shared_tc(flat, shared_up, shared_down)
    out = _combine_sc(ys, sh, pos0, pos1)
    return out.reshape(orig_shape)
